# baseline (speedup 1.0000x reference)
import functools

import jax
import jax.numpy as jnp
from jax import lax
from jax.experimental import pallas as pl
from jax.experimental.pallas import tpu as pltpu
from jax.experimental.pallas import tpu_sc as plsc

B, CDD, HIS, T, H, K = 64, 5, 50, 20, 256, 5
VOCAB = 30000
THRESHOLD = 0.1

CDD_P = 8
HIS_P = 56
SLOTS = CDD_P + HIS_P
NROWS = B * SLOTS
NW = 32
ROWS_PER_W = NROWS // NW
SEGS_PER_CHUNK = 4
TOK_PER_CHUNK = SEGS_PER_CHUNK * T
CHUNKS_PER_W = ROWS_PER_W // SEGS_PER_CHUNK
TOK_PER_W = ROWS_PER_W * T


def _proj_body(x_ref, w_ref, b_ref, o_ref):
    h = jnp.dot(x_ref[...], w_ref[...], preferred_element_type=jnp.float32)
    o_ref[...] = jnp.tanh(h + b_ref[...]) * (1.0 / T)


def _project_table(emb_table, w_enc, b_enc):
    rows = 1200
    return pl.pallas_call(
        _proj_body,
        grid=(VOCAB // rows,),
        in_specs=[
            pl.BlockSpec((rows, H), lambda i: (i, 0)),
            pl.BlockSpec((H, H), lambda i: (0, 0)),
            pl.BlockSpec((1, H), lambda i: (0, 0)),
        ],
        out_specs=pl.BlockSpec((rows, H), lambda i: (i, 0)),
        out_shape=jax.ShapeDtypeStruct((VOCAB, H), jnp.float32),
    )(emb_table, w_enc, b_enc.reshape(1, H))


def _sc_body(tok_hbm, p_hbm, out_hbm, idx_v, rows_v, stage_v, sem):
    wid = lax.axis_index("s") * 2 + lax.axis_index("c")
    pltpu.sync_copy(tok_hbm.at[pl.ds(wid * TOK_PER_W, TOK_PER_W)], idx_v)

    def chunk_body(j, _):
        pltpu.async_copy(
            p_hbm.at[idx_v.at[pl.ds(j * TOK_PER_CHUNK, TOK_PER_CHUNK)]],
            rows_v, sem).wait()
        for k in range(SEGS_PER_CHUNK):
            def t_body(t, accs):
                r = k * T + t
                return tuple(accs[c] + rows_v[r, c * 16:(c + 1) * 16]
                             for c in range(16))
            accs = lax.fori_loop(
                0, T, t_body,
                tuple(jnp.zeros((16,), jnp.float32) for _ in range(16)))
            slot = j * SEGS_PER_CHUNK + k
            for c in range(16):
                stage_v[slot, c * 16:(c + 1) * 16] = accs[c]
        return 0

    lax.fori_loop(0, CHUNKS_PER_W, chunk_body, 0)
    pltpu.sync_copy(stage_v, out_hbm.at[pl.ds(wid * ROWS_PER_W, ROWS_PER_W)])


@functools.partial(
    pl.kernel,
    mesh=plsc.VectorSubcoreMesh(core_axis_name="c", subcore_axis_name="s"),
    out_type=jax.ShapeDtypeStruct((NROWS, H), jnp.float32),
    scratch_types=[
        pltpu.VMEM((TOK_PER_W,), jnp.int32),
        pltpu.VMEM((TOK_PER_CHUNK, H), jnp.float32),
        pltpu.VMEM((ROWS_PER_W, H), jnp.float32),
        pltpu.SemaphoreType.DMA,
    ],
)
def _gather_mean(tok_hbm, p_hbm, out_hbm, idx_v, rows_v, stage_v, sem):
    _sc_body(tok_hbm, p_hbm, out_hbm, idx_v, rows_v, stage_v, sem)


def _head_body(rep_ref, wsel_ref, bsel_ref, wint_ref, bint_ref,
               w1_ref, b1_ref, w2t_ref, o_ref):
    rep_b = rep_ref[...]
    sel = jnp.dot(rep_b, wsel_ref[...],
                  preferred_element_type=jnp.float32) + bsel_ref[...]
    norm = jnp.sqrt(jnp.sum(sel * sel, axis=1, keepdims=True))
    seln = sel / jnp.maximum(norm, 1e-12)
    cddp = seln[0:CDD_P]
    hisp = seln[CDD_P:SLOTS]
    attn = lax.dot_general(cddp, hisp, (((1,), (1,)), ((), ())),
                           preferred_element_type=jnp.float32)

    col = lax.broadcasted_iota(jnp.int32, (CDD_P, HIS_P), 1)
    a = jnp.where(col < HIS, attn, -1e30)
    w = jnp.zeros((CDD_P, HIS_P), jnp.float32)
    for _ in range(K):
        m = jnp.max(a, axis=1, keepdims=True)
        eq = a == m
        first = jnp.min(jnp.where(eq, col, HIS_P), axis=1, keepdims=True)
        onehot = col == first
        w = w + jnp.where(onehot & (m >= THRESHOLD), m, 0.0)
        a = jnp.where(onehot, -1e30, a)

    rvalid = lax.broadcasted_iota(jnp.int32, (HIS_P, H), 0) < HIS
    his_real = jnp.where(rvalid, rep_b[CDD_P:SLOTS], 0.0)
    hisv = jnp.dot(w, his_real,
                   preferred_element_type=jnp.float32) * (1.0 / K)
    fus = jnp.maximum(
        jnp.dot(rep_b[0:CDD_P] * hisv, wint_ref[...],
                preferred_element_type=jnp.float32) + bint_ref[...], 0.0)
    h1 = jnp.maximum(
        jnp.dot(fus, w1_ref[...],
                preferred_element_type=jnp.float32) + b1_ref[...], 0.0)
    score = lax.dot_general(w2t_ref[...], h1, (((1,), (1,)), ((), ())),
                            preferred_element_type=jnp.float32)

    lane = lax.broadcasted_iota(jnp.int32, (1, CDD_P), 1)
    sm = jnp.where(lane < CDD, score, -1e30)
    mx = jnp.max(sm, axis=1, keepdims=True)
    lse = jnp.log(jnp.sum(jnp.exp(sm - mx), axis=1, keepdims=True)) + mx
    o_ref[...] = (score - lse)[None]


def _head(rep, w_sel, b_sel, w_int, b_int, w1, b1, w2):
    return pl.pallas_call(
        _head_body,
        grid=(B,),
        in_specs=[
            pl.BlockSpec((SLOTS, H), lambda b: (b, 0)),
            pl.BlockSpec((H, H), lambda b: (0, 0)),
            pl.BlockSpec((1, H), lambda b: (0, 0)),
            pl.BlockSpec((H, H), lambda b: (0, 0)),
            pl.BlockSpec((1, H), lambda b: (0, 0)),
            pl.BlockSpec((H, H // 2), lambda b: (0, 0)),
            pl.BlockSpec((1, H // 2), lambda b: (0, 0)),
            pl.BlockSpec((1, H // 2), lambda b: (0, 0)),
        ],
        out_specs=pl.BlockSpec((1, 1, CDD_P), lambda b: (b, 0, 0)),
        out_shape=jax.ShapeDtypeStruct((B, 1, CDD_P), jnp.float32),
    )(rep, w_sel, b_sel.reshape(1, H), w_int, b_int.reshape(1, H),
      w1, b1.reshape(1, H // 2), w2.reshape(1, H // 2))


def kernel(cdd_encoded_index, his_encoded_index, emb_table, W_enc, b_enc,
           W_sel, b_sel, W_int, b_int, W1, b1, W2, b2):
    p = _project_table(emb_table, W_enc, b_enc)
    zc = jnp.zeros((B, CDD_P - CDD, T), jnp.int32)
    zh = jnp.zeros((B, HIS_P - HIS, T), jnp.int32)
    tok = jnp.concatenate(
        [cdd_encoded_index.astype(jnp.int32), zc,
         his_encoded_index.astype(jnp.int32), zh], axis=1).reshape(-1)
    rep = _gather_mean(tok, p)
    out = _head(rep, W_sel, b_sel, W_int, b_int, W1, b1, W2)
    return out.reshape(B, CDD_P)[:, :CDD]
